# baseline (speedup 1.0000x reference)
import dataclasses
import functools

import jax
import jax.numpy as jnp
from jax import lax
from jax.experimental import pallas as pl
from jax.experimental.pallas import tpu as pltpu
from jax.experimental.pallas import tpu_sc as plsc

B = 16384
K = 32
L = 16
NC, NS = 2, 16
NW = NC * NS
BPW = B // NW
CH = 128
NCH = BPW // CH


def _bprmf_sc(users, items, Gu, Gi):
  mesh = plsc.VectorSubcoreMesh(core_axis_name="c", subcore_axis_name="s")
  out_type = (
      jax.ShapeDtypeStruct((B,), jnp.float32),
      jax.ShapeDtypeStruct((B, K), jnp.float32),
      jax.ShapeDtypeStruct((B, K), jnp.float32),
  )

  cp = pltpu.CompilerParams(needs_layout_passes=False,
                            use_tc_tiling_on_sc=False)

  @functools.partial(
      pl.kernel,
      mesh=mesh,
      out_type=out_type,
      compiler_params=cp,
      scratch_types=[
          pltpu.VMEM((BPW,), jnp.int32),
          pltpu.VMEM((BPW,), jnp.int32),
          pltpu.VMEM((BPW, K), jnp.float32),
          pltpu.VMEM((BPW, K), jnp.float32),
          pltpu.VMEM((BPW,), jnp.float32),
          pltpu.SemaphoreType.DMA,
          pltpu.SemaphoreType.DMA,
      ],
  )
  def k(users_hbm, items_hbm, gu_tab, gi_tab, xui_hbm, gu_out, gi_out,
        uidx_v, iidx_v, gu_v, gi_v, xui_v, sem_u, sem_i):
    wid = lax.axis_index("s") * NC + lax.axis_index("c")
    base = wid * BPW

    pltpu.sync_copy(users_hbm.at[pl.ds(base, BPW)], uidx_v)
    pltpu.sync_copy(items_hbm.at[pl.ds(base, BPW)], iidx_v)

    copies = []
    for c in range(NCH):
      sl = pl.ds(c * CH, CH)
      copies.append(
          pltpu.async_copy(gu_tab.at[uidx_v.at[sl]], gu_v.at[sl], sem_u))
      copies.append(
          pltpu.async_copy(gi_tab.at[iidx_v.at[sl]], gi_v.at[sl], sem_i))
    for cp in copies:
      cp.wait()

    lane = lax.iota(jnp.int32, L)

    @pl.loop(0, BPW // L)
    def _(g):
      def body(j, v):
        r = g * L + j
        t = (gu_v[r, pl.ds(0, L)] * gi_v[r, pl.ds(0, L)]
             + gu_v[r, pl.ds(L, L)] * gi_v[r, pl.ds(L, L)])
        s = jnp.sum(t)
        return jnp.where(lane == j, s, v)

      xui_v[pl.ds(g * L, L)] = lax.fori_loop(0, L, body,
                                             jnp.zeros((L,), jnp.float32))

    pltpu.sync_copy(xui_v, xui_hbm.at[pl.ds(base, BPW)])
    pltpu.sync_copy(gu_v, gu_out.at[pl.ds(base, BPW)])
    pltpu.sync_copy(gi_v, gi_out.at[pl.ds(base, BPW)])

  return k(users, items, Gu, Gi)


def kernel(users, items, Gu, Gi):
  users = users.astype(jnp.int32)
  items = items.astype(jnp.int32)
  xui, gamma_u, gamma_i = _bprmf_sc(users, items, Gu, Gi)
  return (xui, gamma_u, gamma_i)
